# Initial kernel scaffold; baseline (speedup 1.0000x reference)
import jax
import jax.numpy as jnp
from jax.experimental import pallas as pl


def kernel(idx, tok_emb, pos_emb):
    raise NotImplementedError("write your pallas kernel here")


# speedup vs baseline: 1.3035x; 1.3035x over previous
import functools

import jax
import jax.numpy as jnp
from jax import lax
from jax.experimental import pallas as pl
from jax.experimental.pallas import tpu as pltpu
from jax.experimental.pallas import tpu_sc as plsc

_LANES = 16
_CHUNK = 16
_NBUF = 2


@functools.lru_cache(maxsize=None)
def _build(n_rows, d_model, t_cur):
    info = plsc.get_sparse_core_info()
    nc, ns = info.num_cores, info.num_subcores
    nw = nc * ns
    rpw = n_rows // nw
    c = _CHUNK
    nch = rpw // c
    cw = d_model // _LANES

    mesh = plsc.VectorSubcoreMesh(core_axis_name="c", subcore_axis_name="s")

    @functools.partial(
        pl.kernel,
        mesh=mesh,
        out_type=jax.ShapeDtypeStruct((n_rows, d_model), jnp.float32),
        scratch_types=[
            pltpu.VMEM((rpw,), jnp.int32),
            pltpu.VMEM((_NBUF, c, d_model), jnp.float32),
            pltpu.VMEM((_NBUF, c, d_model), jnp.float32),
            pltpu.SemaphoreType.DMA,
            pltpu.SemaphoreType.DMA,
            pltpu.SemaphoreType.DMA,
            pltpu.SemaphoreType.DMA,
        ],
    )
    def emb_kernel(idx_hbm, tok_hbm, pos_hbm, out_hbm,
                   idx_v, rows_v, pos_v, sg0, sg1, sp0, sp1):
        wid = lax.axis_index("s") * nc + lax.axis_index("c")
        base = wid * rpw
        tbase = base % t_cur
        pltpu.sync_copy(idx_hbm.at[pl.ds(base, rpw)], idx_v)

        sgs = (sg0, sg1)
        sps = (sp0, sp1)
        gcp = [None] * _NBUF
        pcp = [None] * _NBUF

        def issue(g):
            b = g % _NBUF
            gcp[b] = pltpu.async_copy(
                tok_hbm.at[idx_v.at[pl.ds(g * c, c)]], rows_v.at[b], sgs[b])
            pcp[b] = pltpu.async_copy(
                pos_hbm.at[pl.ds(tbase + g * c, c)], pos_v.at[b], sps[b])

        for g in range(min(_NBUF, nch)):
            issue(g)

        for g in range(nch):
            b = g % _NBUF
            gcp[b].wait()
            pcp[b].wait()

            def add_body(i, carry, _b=b):
                r = i // cw
                col = (i % cw) * _LANES
                x = pos_v[_b, r, pl.ds(col, _LANES)]
                plsc.addupdate(rows_v.at[_b, r, pl.ds(col, _LANES)], x)
                return carry

            lax.fori_loop(0, c * cw, add_body, 0, unroll=8)
            pltpu.sync_copy(rows_v.at[b], out_hbm.at[pl.ds(base + g * c, c)])
            if g + _NBUF < nch:
                issue(g + _NBUF)

    return emb_kernel


def kernel(idx, tok_emb, pos_emb):
    b, t_cur = idx.shape
    d_model = tok_emb.shape[1]
    n_rows = b * t_cur
    flat_idx = idx.reshape(n_rows).astype(jnp.int32)
    out = _build(n_rows, d_model, t_cur)(flat_idx, tok_emb, pos_emb)
    return out.reshape(b, t_cur, d_model)
